# baseline (speedup 1.0000x reference)
import functools

import jax
import jax.numpy as jnp
from jax import lax
from jax.experimental import pallas as pl
from jax.experimental.pallas import tpu as pltpu
from jax.experimental.pallas import tpu_sc as plsc

_B, _C, _D = 128, 1024, 256
_NC, _NS = 2, 16
_NW = _NC * _NS
_BPW = _B // _NW
_CHUNK = 64
_CPB = _C // _CHUNK
_NBUF = 4
_T = _BPW * _CPB
_NGRP = _T // _NBUF
_PAIR = 2 * _CPB


def _worker_body(x_hbm, idx_hbm, inv_hbm, out_hbm, idx_v, inv1_v, inv_v,
                 rows_v, gsems, wsems):
    wid = lax.axis_index("s") * _NC + lax.axis_index("c")
    pltpu.sync_copy(idx_hbm, idx_v)
    pltpu.sync_copy(inv_hbm, inv1_v)
    b0 = wid * _BPW

    def repack_body(j, carry):
        src = pl.multiple_of(j * 16, 16)
        col = pl.multiple_of((j % (_CHUNK // 16)) * 16, 16)
        inv_v[j // (_CHUNK // 16), pl.ds(col, 16)] = inv1_v[pl.ds(src, 16)]
        return carry

    lax.fori_loop(0, _C // 16, repack_body, 0)

    def read(t, slot, mode):
        p = t // _PAIR
        k = (t % _PAIR) // 2
        b = b0 + 2 * p + mode
        koff = pl.multiple_of(k * _CHUNK, _CHUNK)
        if mode == 0:
            src = x_hbm.at[b].at[idx_v.at[pl.ds(koff, _CHUNK)]]
        else:
            src = x_hbm.at[b].at[pl.ds(koff, _CHUNK)]
        pltpu.async_copy(src, rows_v.at[slot], gsems[slot])

    def wait_read(slot):
        pltpu.make_async_copy(
            x_hbm.at[0].at[pl.ds(0, _CHUNK)],
            rows_v.at[slot],
            gsems[slot],
        ).wait()

    def write(t, slot, mode):
        p = t // _PAIR
        k = (t % _PAIR) // 2
        b = b0 + 2 * p + mode
        if mode == 0:
            koff = pl.multiple_of(k * _CHUNK, _CHUNK)
            dst = out_hbm.at[b].at[pl.ds(koff, _CHUNK)]
        else:
            dst = out_hbm.at[b].at[inv_v.at[k]]
        pltpu.async_copy(rows_v.at[slot], dst, wsems[slot])

    def wait_write(slot):
        pltpu.make_async_copy(
            rows_v.at[slot],
            out_hbm.at[0].at[pl.ds(0, _CHUNK)],
            wsems[slot],
        ).wait()

    def group_body(g, carry):
        t0 = g * _NBUF
        for s in range(_NBUF):
            mode = s % 2

            @pl.when(g > 0)
            def _():
                wait_write(s)

            read(t0 + s, s, mode)
            prev = (s - 2) % _NBUF
            prev_mode = prev % 2
            if s <= 1:

                @pl.when(g > 0)
                def _():
                    wait_read(prev)
                    write(t0 + s - 2, prev, prev_mode)

            else:
                wait_read(prev)
                write(t0 + s - 2, prev, prev_mode)
        return carry

    lax.fori_loop(0, _NGRP, group_body, 0)
    for s in (_NBUF - 2, _NBUF - 1):
        wait_read(s)
        write(_T - _NBUF + s, s, s % 2)
    for s in range(_NBUF):
        wait_write(s)


@jax.jit
def _sc_subsample(x, idx, inv):
    mesh = plsc.VectorSubcoreMesh(core_axis_name="c", subcore_axis_name="s")
    f = pl.kernel(
        _worker_body,
        mesh=mesh,
        out_type=jax.ShapeDtypeStruct((_B, _C, _D), jnp.float32),
        scratch_types=[
            pltpu.VMEM((_C,), jnp.int32),
            pltpu.VMEM((_C,), jnp.int32),
            pltpu.VMEM((_CPB, _CHUNK), jnp.int32),
            pltpu.VMEM((_NBUF, _CHUNK, _D), jnp.float32),
            [pltpu.SemaphoreType.DMA] * _NBUF,
            [pltpu.SemaphoreType.DMA] * _NBUF,
        ],
    )
    return f(x, idx, inv)


def kernel(x, forward_shuffle_idx):
    inv = (
        jnp.zeros((_C,), jnp.int32)
        .at[forward_shuffle_idx]
        .set(jnp.arange(_C, dtype=jnp.int32))
    )
    return _sc_subsample(x, forward_shuffle_idx, inv)
